# baseline (speedup 1.0000x reference)
import jax
import jax.numpy as jnp
from jax.experimental import pallas as pl
from jax.experimental.pallas import tpu as pltpu

_K = 8
_BLOCK = 2000


def _decoder_kernel(x_ref, wg1_ref, bg1_ref, wdec_ref, bdec_ref,
                    w1a_ref, w1b_ref, w1c_ref, b1_ref,
                    rel_ref, dec_ref):
    b = x_ref.shape[0]
    x = x_ref[...]
    nbr = jnp.maximum(
        jnp.dot(x, wg1_ref[...], preferred_element_type=jnp.float32)
        + bg1_ref[...], 0.0)
    relraw = (jnp.dot(nbr, wdec_ref[...], preferred_element_type=jnp.float32)
              + bdec_ref[...])
    rel3 = relraw.reshape(b, _K, 3)
    rel_ref[...] = rel3
    base = (jnp.dot(x, w1a_ref[...], preferred_element_type=jnp.float32)
            + jnp.dot(nbr, w1b_ref[...], preferred_element_type=jnp.float32)
            + b1_ref[...])
    contrib = jax.lax.dot_general(
        rel3, w1c_ref[...], (((2,), (0,)), ((), ())),
        preferred_element_type=jnp.float32)
    dec_ref[...] = jnp.maximum(base[:, None, :] + contrib, 0.0)


def kernel(input_features, W_g1, b_g1, W_dec, b_dec, W_1, b_1):
    n, d = input_features.shape
    k = _K
    h = W_1.shape[1]
    W_1a = W_1[:d]
    W_1b = W_1[d:2 * d]
    W_1c = W_1[2 * d:]

    block = _BLOCK
    grid = (n // block,)
    full = lambda i: (0, 0)
    row_blocked = lambda i: (i, 0)
    row_blocked3 = lambda i: (i, 0, 0)

    rel3, dec3 = pl.pallas_call(
        _decoder_kernel,
        grid=grid,
        in_specs=[
            pl.BlockSpec((block, d), row_blocked),
            pl.BlockSpec(W_g1.shape, full),
            pl.BlockSpec((1, W_g1.shape[1]), full),
            pl.BlockSpec(W_dec.shape, full),
            pl.BlockSpec((1, W_dec.shape[1]), full),
            pl.BlockSpec(W_1a.shape, full),
            pl.BlockSpec(W_1b.shape, full),
            pl.BlockSpec(W_1c.shape, full),
            pl.BlockSpec((1, h), full),
        ],
        out_specs=[
            pl.BlockSpec((block, k, 3), row_blocked3),
            pl.BlockSpec((block, k, h), row_blocked3),
        ],
        out_shape=[
            jax.ShapeDtypeStruct((n, k, 3), jnp.float32),
            jax.ShapeDtypeStruct((n, k, h), jnp.float32),
        ],
        compiler_params=pltpu.CompilerParams(
            dimension_semantics=("arbitrary",),
        ),
    )(input_features, W_g1, b_g1.reshape(1, -1), W_dec,
      b_dec.reshape(1, -1), W_1a, W_1b, W_1c, b_1.reshape(1, -1))

    relative_points = rel3.reshape(n * k, 3)
    decoded_features = dec3.reshape(n * k, h)
    cluster = jnp.repeat(jnp.arange(n, dtype=jnp.int32), k)
    return (relative_points, decoded_features, cluster)


# speedup vs baseline: 1.0454x; 1.0454x over previous
import functools

import jax
import jax.numpy as jnp
from jax import lax
from jax.experimental import pallas as pl
from jax.experimental.pallas import tpu as pltpu
from jax.experimental.pallas import tpu_sc as plsc

_K = 8
_BLOCK = 2000

_SC_CORES = 2
_SC_SUBCORES = 16
_SC_LANES = 16


def _cluster_on_sparsecore(total):
    workers = 25
    per_w = total // workers

    @functools.partial(
        pl.kernel,
        mesh=plsc.VectorSubcoreMesh(core_axis_name="c", subcore_axis_name="s"),
        out_type=jax.ShapeDtypeStruct((total,), jnp.int32),
        scratch_types=[pltpu.VMEM((per_w,), jnp.int32)],
    )
    def cluster_kernel(out_hbm, buf):
        wid = lax.axis_index("s") * _SC_CORES + lax.axis_index("c")

        @pl.when(wid < workers)
        def _():
            base = wid * per_w

            def body(i, carry):
                start = i * _SC_LANES
                vals = (base + start + lax.iota(jnp.int32, _SC_LANES)) >> 3
                buf[pl.ds(start, _SC_LANES)] = vals
                return carry

            lax.fori_loop(0, per_w // _SC_LANES, body, 0)
            pltpu.sync_copy(buf, out_hbm.at[pl.ds(base, per_w)])

    return cluster_kernel()


def _decoder_kernel(x_ref, wg1_ref, bg1_ref, wdec_ref, bdec_ref,
                    w1a_ref, w1b_ref, w1c_ref, b1_ref,
                    rel_ref, dec_ref):
    b = x_ref.shape[0]
    x = x_ref[...]
    nbr = jnp.maximum(
        jnp.dot(x, wg1_ref[...], preferred_element_type=jnp.float32)
        + bg1_ref[...], 0.0)
    relraw = (jnp.dot(nbr, wdec_ref[...], preferred_element_type=jnp.float32)
              + bdec_ref[...])
    rel3 = relraw.reshape(b, _K, 3)
    rel_ref[...] = rel3
    base = (jnp.dot(x, w1a_ref[...], preferred_element_type=jnp.float32)
            + jnp.dot(nbr, w1b_ref[...], preferred_element_type=jnp.float32)
            + b1_ref[...])
    contrib = jax.lax.dot_general(
        rel3, w1c_ref[...], (((2,), (0,)), ((), ())),
        preferred_element_type=jnp.float32)
    dec_ref[...] = jnp.maximum(base[:, None, :] + contrib, 0.0)


def kernel(input_features, W_g1, b_g1, W_dec, b_dec, W_1, b_1):
    n, d = input_features.shape
    k = _K
    h = W_1.shape[1]
    W_1a = W_1[:d]
    W_1b = W_1[d:2 * d]
    W_1c = W_1[2 * d:]

    cluster = _cluster_on_sparsecore(n * k)

    block = _BLOCK
    grid = (n // block,)
    full = lambda i: (0, 0)
    row_blocked = lambda i: (i, 0)
    row_blocked3 = lambda i: (i, 0, 0)

    rel3, dec3 = pl.pallas_call(
        _decoder_kernel,
        grid=grid,
        in_specs=[
            pl.BlockSpec((block, d), row_blocked),
            pl.BlockSpec(W_g1.shape, full),
            pl.BlockSpec((1, W_g1.shape[1]), full),
            pl.BlockSpec(W_dec.shape, full),
            pl.BlockSpec((1, W_dec.shape[1]), full),
            pl.BlockSpec(W_1a.shape, full),
            pl.BlockSpec(W_1b.shape, full),
            pl.BlockSpec(W_1c.shape, full),
            pl.BlockSpec((1, h), full),
        ],
        out_specs=[
            pl.BlockSpec((block, k, 3), row_blocked3),
            pl.BlockSpec((block, k, h), row_blocked3),
        ],
        out_shape=[
            jax.ShapeDtypeStruct((n, k, 3), jnp.float32),
            jax.ShapeDtypeStruct((n, k, h), jnp.float32),
        ],
        compiler_params=pltpu.CompilerParams(
            dimension_semantics=("arbitrary",),
        ),
    )(input_features, W_g1, b_g1.reshape(1, -1), W_dec,
      b_dec.reshape(1, -1), W_1a, W_1b, W_1c, b_1.reshape(1, -1))

    relative_points = rel3.reshape(n * k, 3)
    decoded_features = dec3.reshape(n * k, h)
    return (relative_points, decoded_features, cluster)


# speedup vs baseline: 1.0459x; 1.0005x over previous
import functools

import jax
import jax.numpy as jnp
from jax import lax
from jax.experimental import pallas as pl
from jax.experimental.pallas import tpu as pltpu
from jax.experimental.pallas import tpu_sc as plsc

_K = 8
_BLOCK = 2000

_SC_CORES = 2
_SC_SUBCORES = 16
_SC_LANES = 16


def _cluster_on_sparsecore(total):
    workers = 25
    per_w = total // workers

    @functools.partial(
        pl.kernel,
        mesh=plsc.VectorSubcoreMesh(core_axis_name="c", subcore_axis_name="s"),
        out_type=jax.ShapeDtypeStruct((total,), jnp.int32),
        scratch_types=[pltpu.VMEM((per_w,), jnp.int32)],
    )
    def cluster_kernel(out_hbm, buf):
        wid = lax.axis_index("s") * _SC_CORES + lax.axis_index("c")

        @pl.when(wid < workers)
        def _():
            base = wid * per_w

            def body(i, carry):
                start = i * _SC_LANES
                vals = (base + start + lax.iota(jnp.int32, _SC_LANES)) >> 3
                buf[pl.ds(start, _SC_LANES)] = vals
                return carry

            lax.fori_loop(0, per_w // _SC_LANES, body, 0)
            pltpu.sync_copy(buf, out_hbm.at[pl.ds(base, per_w)])

    return cluster_kernel()


def _decoder_kernel(x_ref, wg1_ref, bg1_ref, wdec_ref, bdec_ref,
                    w1a_ref, w1b_ref, w1c_ref, b1_ref,
                    rel_ref, dec_ref):
    b = x_ref.shape[0]
    x = x_ref[...]
    nbr = jnp.maximum(
        jnp.dot(x, wg1_ref[...], preferred_element_type=jnp.float32)
        + bg1_ref[...], 0.0)
    relraw = (jnp.dot(nbr, wdec_ref[...], preferred_element_type=jnp.float32)
              + bdec_ref[...])
    rel3 = relraw.reshape(b, _K, 3)
    rel_ref[...] = rel3
    base = (jnp.dot(x, w1a_ref[...], preferred_element_type=jnp.float32)
            + jnp.dot(nbr, w1b_ref[...], preferred_element_type=jnp.float32)
            + b1_ref[...])
    contrib = jax.lax.dot_general(
        rel3, w1c_ref[...], (((2,), (0,)), ((), ())),
        preferred_element_type=jnp.float32)
    dec_ref[...] = jnp.maximum(base[:, None, :] + contrib, 0.0)


def kernel(input_features, W_g1, b_g1, W_dec, b_dec, W_1, b_1):
    n, d = input_features.shape
    k = _K
    h = W_1.shape[1]
    W_1a = W_1[:d]
    W_1b = W_1[d:2 * d]
    W_1c = W_1[2 * d:]

    cluster = _cluster_on_sparsecore(n * k)

    block = _BLOCK
    grid = (n // block,)
    full = lambda i: (0, 0)
    row_blocked = lambda i: (i, 0)
    row_blocked3 = lambda i: (i, 0, 0)

    rel3, dec3 = pl.pallas_call(
        _decoder_kernel,
        grid=grid,
        in_specs=[
            pl.BlockSpec((block, d), row_blocked),
            pl.BlockSpec(W_g1.shape, full),
            pl.BlockSpec((1, W_g1.shape[1]), full),
            pl.BlockSpec(W_dec.shape, full),
            pl.BlockSpec((1, W_dec.shape[1]), full),
            pl.BlockSpec(W_1a.shape, full),
            pl.BlockSpec(W_1b.shape, full),
            pl.BlockSpec(W_1c.shape, full),
            pl.BlockSpec((1, h), full),
        ],
        out_specs=[
            pl.BlockSpec((block, k, 3), row_blocked3),
            pl.BlockSpec((block, k, h), row_blocked3),
        ],
        out_shape=[
            jax.ShapeDtypeStruct((n, k, 3), jnp.float32),
            jax.ShapeDtypeStruct((n, k, h), jnp.float32),
        ],
        compiler_params=pltpu.CompilerParams(
            dimension_semantics=("parallel",),
        ),
    )(input_features, W_g1, b_g1.reshape(1, -1), W_dec,
      b_dec.reshape(1, -1), W_1a, W_1b, W_1c, b_1.reshape(1, -1))

    relative_points = rel3.reshape(n * k, 3)
    decoded_features = dec3.reshape(n * k, h)
    return (relative_points, decoded_features, cluster)
